# baseline (speedup 1.0000x reference)
import functools

import jax
import jax.numpy as jnp
from jax import lax
from jax.experimental import pallas as pl
from jax.experimental.pallas import tpu as pltpu
from jax.experimental.pallas import tpu_sc as plsc


def kernel(center_ids, context_ids, center_weight, center_biase, context_weight, context_biase):
    del context_ids
    B = center_ids.shape[0]
    V, D = center_weight.shape
    L = 16
    W = 4 * D

    info = plsc.get_sparse_core_info()
    NC, NS = info.num_cores, info.num_subcores
    NW = NC * NS
    n = B // NW
    C = 256
    n_chunks = n // C

    ids = center_ids.astype(jnp.int32)

    def repack(w):
        wT = w.T
        G = 512
        nblk = (V + G - 1) // G

        def body(x_ref, o_ref):
            x = x_ref[...]
            o_ref[...] = jnp.concatenate(
                [x[:, d * 128:(d + 1) * 128].T for d in range(4)], axis=1)

        return pl.pallas_call(
            body,
            grid=(nblk,),
            in_specs=[pl.BlockSpec((D, G), lambda R: (0, R))],
            out_specs=pl.BlockSpec((G // 4, W), lambda R: (R, 0)),
            out_shape=jax.ShapeDtypeStruct((nblk * (G // 4), W), jnp.float32),
        )(wT)

    cw2 = repack(center_weight)
    xw2 = repack(context_weight)
    cbf = center_biase.reshape(V)
    xbf = context_biase.reshape(V)
    mesh = plsc.VectorSubcoreMesh(core_axis_name="c", subcore_axis_name="s")

    @functools.partial(
        pl.kernel,
        mesh=mesh,
        compiler_params=pltpu.CompilerParams(
            needs_layout_passes=False,
        ),
        out_type=jax.ShapeDtypeStruct((B,), jnp.float32),
        scratch_types=[
            pltpu.VMEM((n,), jnp.int32),
            pltpu.VMEM((n,), jnp.int32),
            pltpu.VMEM((C, W), jnp.float32),
            pltpu.VMEM((C, W), jnp.float32),
            pltpu.VMEM((n,), jnp.float32),
            pltpu.VMEM((n,), jnp.float32),
            pltpu.VMEM((n,), jnp.float32),
            pltpu.SemaphoreType.DMA,
        ],
    )
    def glove_sc(ids_hbm, cw_hbm, cb_hbm, xw_hbm, xb_hbm, out_hbm,
                 idx_v, q_v, cwb, xwb, cbb, xbb, out_v, sem):
        wid = lax.axis_index("s") * NC + lax.axis_index("c")
        base = wid * n

        pltpu.sync_copy(ids_hbm.at[pl.ds(base, n)], idx_v)
        lanes = lax.iota(jnp.int32, L)

        def blocks(g):
            iv = idx_v[pl.ds(g * L, L)]
            q_v[pl.ds(g * L, L)] = (iv >> 9) * 128 + (iv & 127)

        pl.loop(0, n // L)(blocks)

        g3 = pltpu.async_copy(cb_hbm.at[idx_v], cbb, sem)
        g4 = pltpu.async_copy(xb_hbm.at[idx_v], xbb, sem)

        for c in range(n_chunks):
            g1 = pltpu.async_copy(cw_hbm.at[q_v.at[pl.ds(c * C, C)]], cwb, sem)
            g2 = pltpu.async_copy(xw_hbm.at[q_v.at[pl.ds(c * C, C)]], xwb, sem)
            g1.wait()
            g2.wait()
            if c == 0:
                g3.wait()
                g4.wait()

            def comp(t):
                o = t * L
                acc = cbb[pl.ds(c * C + o, L)] + xbb[pl.ds(c * C + o, L)]
                iv = idx_v[pl.ds(c * C + o, L)]
                for k in range(L):
                    col = ((iv[k] >> 7) & 3) * D
                    cols = col + lanes
                    rowv = jnp.full((L,), o + k, dtype=jnp.int32)
                    p = plsc.load_gather(cwb, [rowv, cols]) * plsc.load_gather(xwb, [rowv, cols])
                    p = p + plsc.load_gather(cwb, [rowv, cols + L]) * plsc.load_gather(xwb, [rowv, cols + L])
                    s = jnp.sum(p)
                    acc = acc + jnp.where(lanes == k, s, jnp.float32(0.0))
                out_v[pl.ds(c * C + o, L)] = acc

            pl.loop(0, C // L)(comp)

        pltpu.sync_copy(out_v, out_hbm.at[pl.ds(base, n)])

    out = glove_sc(ids, cw2, cbf, xw2, xbf)
    return out.reshape(B, 1)


# speedup vs baseline: 2.6820x; 2.6820x over previous
import functools

import jax
import jax.numpy as jnp
from jax import lax
from jax.experimental import pallas as pl
from jax.experimental.pallas import tpu as pltpu
from jax.experimental.pallas import tpu_sc as plsc


def kernel(center_ids, context_ids, center_weight, center_biase, context_weight, context_biase):
    del context_ids
    B = center_ids.shape[0]
    V, D = center_weight.shape
    L = 16
    W = 4 * D

    info = plsc.get_sparse_core_info()
    NC, NS = info.num_cores, info.num_subcores
    NW = NC * NS
    n = B // NW
    C = 256
    n_chunks = n // C

    ids = center_ids.astype(jnp.int32)
    cw2 = center_weight.reshape(V // 4, W)
    xw2 = context_weight.reshape(V // 4, W)
    cbf = center_biase.reshape(V)
    xbf = context_biase.reshape(V)
    mesh = plsc.VectorSubcoreMesh(core_axis_name="c", subcore_axis_name="s")

    @functools.partial(
        pl.kernel,
        mesh=mesh,
        compiler_params=pltpu.CompilerParams(
            needs_layout_passes=False,
        ),
        out_type=jax.ShapeDtypeStruct((B,), jnp.float32),
        scratch_types=[
            pltpu.VMEM((n,), jnp.int32),
            pltpu.VMEM((n,), jnp.int32),
            pltpu.VMEM((C, W), jnp.float32),
            pltpu.VMEM((C, W), jnp.float32),
            pltpu.VMEM((n,), jnp.float32),
            pltpu.VMEM((n,), jnp.float32),
            pltpu.VMEM((n,), jnp.float32),
            pltpu.SemaphoreType.DMA,
        ],
    )
    def glove_sc(ids_hbm, cw_hbm, cb_hbm, xw_hbm, xb_hbm, out_hbm,
                 idx_v, q_v, cwb, xwb, cbb, xbb, out_v, sem):
        wid = lax.axis_index("s") * NC + lax.axis_index("c")
        base = wid * n

        pltpu.sync_copy(ids_hbm.at[pl.ds(base, n)], idx_v)
        lanes = lax.iota(jnp.int32, L)

        def blocks(g):
            iv = idx_v[pl.ds(g * L, L)]
            q_v[pl.ds(g * L, L)] = iv >> 2

        pl.loop(0, n // L)(blocks)

        g3 = pltpu.async_copy(cb_hbm.at[idx_v], cbb, sem)
        g4 = pltpu.async_copy(xb_hbm.at[idx_v], xbb, sem)

        for c in range(n_chunks):
            g1 = pltpu.async_copy(cw_hbm.at[q_v.at[pl.ds(c * C, C)]], cwb, sem)
            g2 = pltpu.async_copy(xw_hbm.at[q_v.at[pl.ds(c * C, C)]], xwb, sem)
            g1.wait()
            g2.wait()
            if c == 0:
                g3.wait()
                g4.wait()

            def comp(t):
                o = t * L
                acc = cbb[pl.ds(c * C + o, L)] + xbb[pl.ds(c * C + o, L)]
                iv = idx_v[pl.ds(c * C + o, L)]
                for k in range(L):
                    col = (iv[k] & 3) * D
                    cols = col + lanes
                    rowv = jnp.full((L,), o + k, dtype=jnp.int32)
                    p = plsc.load_gather(cwb, [rowv, cols]) * plsc.load_gather(xwb, [rowv, cols])
                    p = p + plsc.load_gather(cwb, [rowv, cols + L]) * plsc.load_gather(xwb, [rowv, cols + L])
                    s = jnp.sum(p)
                    acc = acc + jnp.where(lanes == k, s, jnp.float32(0.0))
                out_v[pl.ds(c * C + o, L)] = acc

            pl.loop(0, C // L)(comp)

        pltpu.sync_copy(out_v, out_hbm.at[pl.ds(base, n)])

    out = glove_sc(ids, cw2, cbf, xw2, xbf)
    return out.reshape(B, 1)
